# baseline (speedup 1.0000x reference)
import jax
import jax.numpy as jnp
from jax import lax
from jax.experimental import pallas as pl
from jax.experimental.pallas import tpu as pltpu
from jax.experimental.pallas import tpu_sc as plsc

NUM_FIELDS = 26
VOCAB = 100000
EMBED_DIM = 32
BATCH = 16384
SCALE = NUM_FIELDS ** -0.5

_info = plsc.get_sparse_core_info()
NC, NS, L = _info.num_cores, _info.num_subcores, _info.num_lanes
NW = NC * NS
H0 = 50048
H1 = VOCAB - H0
TAIL = VOCAB % 128
H1MAIN = H1 - TAIL
CLAMPS = (H0, H1)
NK = 2 * NUM_FIELDS
IC = 4096
NIC = BATCH // IC
UNROLL = 1


def _body(xt_hbm, tabT_hbm, tail_hbm, outT_hbm, idx0_v, idx1_v,
          row0_v, row1_v, acc_v, rsem0, rsem1, isem0, isem1):
    wid = lax.axis_index("s") * NC + lax.axis_index("c")
    rows = (row0_v, row1_v)
    idxs = (idx0_v, idx1_v)
    rsems = (rsem0, rsem1)
    isems = (isem0, isem1)

    zeros = jnp.zeros((L,), jnp.float32)
    row0_v[pl.ds(H0, L)] = zeros

    @plsc.parallel_loop(0, BATCH, step=L, unroll=2)
    def _zero(i):
        acc_v[pl.ds(i, L)] = zeros

    def row_start(k):
        f, h = k // 2, k % 2
        if h == 0:
            return [pltpu.async_copy(
                tabT_hbm.at[f, wid, pl.ds(0, H0)],
                row0_v.at[pl.ds(0, H0)], rsems[0])]
        return [
            pltpu.async_copy(
                tabT_hbm.at[f, wid, pl.ds(H0, H1MAIN)],
                row1_v.at[pl.ds(0, H1MAIN)], rsems[1]),
            pltpu.async_copy(
                tail_hbm.at[f, wid], row1_v.at[pl.ds(H1MAIN, 128)],
                rsems[1]),
        ]

    def idx_start(k, c):
        return pltpu.async_copy(
            xt_hbm.at[k // 2, pl.ds(c * IC, IC)], idxs[c % 2],
            isems[c % 2])

    rd = [row_start(0), row_start(1)]
    idn = [idx_start(0, 0), idx_start(0, 1)]

    for k in range(NK):
        h = k % 2
        for d in rd[h]:
            d.wait()
        for c in range(NIC):
            idn[c % 2].wait()
            if c + 2 < NIC:
                idn[c % 2] = idx_start(k, c + 2)
            elif k + 1 < NK:
                idn[c % 2] = idx_start(k + 1, c + 2 - NIC)
            ib = c % 2
            cbase = c * IC

            @plsc.parallel_loop(0, IC, step=L, unroll=UNROLL)
            def _gather(i):
                iv = idxs[ib][pl.ds(i, L)]
                if h:
                    iv = iv - H0
                ivc = jnp.minimum(plsc.bitcast(iv, jnp.uint32),
                                  jnp.uint32(CLAMPS[h]))
                vals = plsc.load_gather(rows[h],
                                        [plsc.bitcast(ivc, jnp.int32)])
                plsc.addupdate(acc_v.at[pl.ds(cbase + i, L)], vals)

        if k + 2 < NK:
            rd[h] = row_start(k + 2)

    @plsc.parallel_loop(0, BATCH, step=L, unroll=2)
    def _scale(i):
        sl = pl.ds(i, L)
        acc_v[sl] = acc_v[sl] * SCALE

    pltpu.sync_copy(acc_v, outT_hbm.at[wid])


def _embed_sum(xt, tabT, tail):
    mesh = plsc.VectorSubcoreMesh(core_axis_name="c", subcore_axis_name="s")
    return pl.kernel(
        _body,
        out_type=jax.ShapeDtypeStruct((EMBED_DIM, BATCH), jnp.float32),
        mesh=mesh,
        scratch_types=[
            pltpu.VMEM((IC,), jnp.int32),
            pltpu.VMEM((IC,), jnp.int32),
            pltpu.VMEM((H0 + L,), jnp.float32),
            pltpu.VMEM((H1MAIN + 128,), jnp.float32),
            pltpu.VMEM((BATCH,), jnp.float32),
            pltpu.SemaphoreType.DMA,
            pltpu.SemaphoreType.DMA,
            pltpu.SemaphoreType.DMA,
            pltpu.SemaphoreType.DMA,
        ],
        compiler_params=pltpu.CompilerParams(needs_layout_passes=False),
    )(xt, tabT, tail)


def kernel(x, tables):
    if x.ndim == 1:
        x = x[:, None]
    xt = x.T
    tabT = tables.transpose(0, 2, 1)
    tail = jnp.pad(tables[:, VOCAB - TAIL:, :].transpose(0, 2, 1),
                   ((0, 0), (0, 0), (0, 128 - TAIL)))
    outT = _embed_sum(xt, tabT, tail)
    return outT.T


# speedup vs baseline: 1.2340x; 1.2340x over previous
import jax
import jax.numpy as jnp
from jax import lax
from jax.experimental import pallas as pl
from jax.experimental.pallas import tpu as pltpu
from jax.experimental.pallas import tpu_sc as plsc

NUM_FIELDS = 26
VOCAB = 100000
EMBED_DIM = 32
BATCH = 16384
SCALE = NUM_FIELDS ** -0.5

_info = plsc.get_sparse_core_info()
NC, NS, L = _info.num_cores, _info.num_subcores, _info.num_lanes
NW = NC * NS
H0 = 50048
H1 = VOCAB - H0
TAIL = VOCAB % 128
H1MAIN = H1 - TAIL
CLAMPS = (H0, H1)
IC = 4096
NIC = BATCH // IC
UNROLL = 4


def _body(xt_hbm, tabT_hbm, tail_hbm, outT_hbm, idx0_v, idx1_v,
          row0_v, row1_v, acc_v, rsem0, rsem1, isem0, isem1):
    wid = lax.axis_index("s") * NC + lax.axis_index("c")
    idxs = (idx0_v, idx1_v)
    isems = (isem0, isem1)

    zeros = jnp.zeros((L,), jnp.float32)
    row0_v[pl.ds(H0, L)] = zeros

    @plsc.parallel_loop(0, BATCH, step=L, unroll=2)
    def _zero(i):
        acc_v[pl.ds(i, L)] = zeros

    def row_descs(f, h):
        if h == 0:
            return [pltpu.make_async_copy(
                tabT_hbm.at[f, wid, pl.ds(0, H0)],
                row0_v.at[pl.ds(0, H0)], rsem0)]
        return [
            pltpu.make_async_copy(
                tabT_hbm.at[f, wid, pl.ds(H0, H1MAIN)],
                row1_v.at[pl.ds(0, H1MAIN)], rsem1),
            pltpu.make_async_copy(
                tail_hbm.at[f, wid], row1_v.at[pl.ds(H1MAIN, 128)], rsem1),
        ]

    def row_issue(f, h):
        for d in row_descs(f, h):
            d.start()

    def row_wait(h):
        for d in row_descs(0, h):
            d.wait()

    def idx_issue(f, c):
        pltpu.make_async_copy(
            xt_hbm.at[f, pl.ds(c * IC, IC)], idxs[c % 2], isems[c % 2]
        ).start()

    def idx_wait(c):
        pltpu.make_async_copy(
            xt_hbm.at[0, pl.ds(0, IC)], idxs[c % 2], isems[c % 2]).wait()

    row_issue(0, 0)
    row_issue(0, 1)
    idx_issue(0, 0)
    idx_issue(0, 1)

    @pl.loop(0, NUM_FIELDS)
    def _field(f):
        not_last = f < NUM_FIELDS - 1
        for h in (0, 1):
            row_wait(h)
            for c in range(NIC):
                idx_wait(c)
                ib = c % 2
                cbase = c * IC

                @plsc.parallel_loop(0, IC, step=L, unroll=UNROLL)
                def _gather(i):
                    iv = idxs[ib][pl.ds(i, L)]
                    if h:
                        iv = iv - H0
                    ivc = jnp.minimum(plsc.bitcast(iv, jnp.uint32),
                                      jnp.uint32(CLAMPS[h]))
                    vals = plsc.load_gather(
                        (row0_v, row1_v)[h],
                        [plsc.bitcast(ivc, jnp.int32)])
                    plsc.addupdate(acc_v.at[pl.ds(cbase + i, L)], vals)

                if c + 2 < NIC:
                    idx_issue(f, c + 2)
                elif h == 0:
                    idx_issue(f, c + 2 - NIC)
                else:
                    @pl.when(not_last)
                    def _pre_idx():
                        idx_issue(f + 1, c + 2 - NIC)

            @pl.when(not_last)
            def _pre_row():
                row_issue(f + 1, h)

    @plsc.parallel_loop(0, BATCH, step=L, unroll=2)
    def _scale(i):
        sl = pl.ds(i, L)
        acc_v[sl] = acc_v[sl] * SCALE

    pltpu.sync_copy(acc_v, outT_hbm.at[wid])


def _embed_sum(xt, tabT, tail):
    mesh = plsc.VectorSubcoreMesh(core_axis_name="c", subcore_axis_name="s")
    return pl.kernel(
        _body,
        out_type=jax.ShapeDtypeStruct((EMBED_DIM, BATCH), jnp.float32),
        mesh=mesh,
        scratch_types=[
            pltpu.VMEM((IC,), jnp.int32),
            pltpu.VMEM((IC,), jnp.int32),
            pltpu.VMEM((H0 + L,), jnp.float32),
            pltpu.VMEM((H1MAIN + 128,), jnp.float32),
            pltpu.VMEM((BATCH,), jnp.float32),
            pltpu.SemaphoreType.DMA,
            pltpu.SemaphoreType.DMA,
            pltpu.SemaphoreType.DMA,
            pltpu.SemaphoreType.DMA,
        ],
        compiler_params=pltpu.CompilerParams(needs_layout_passes=False),
    )(xt, tabT, tail)


def kernel(x, tables):
    if x.ndim == 1:
        x = x[:, None]
    xt = x.T
    tabT = tables.transpose(0, 2, 1)
    tail = jnp.pad(tables[:, VOCAB - TAIL:, :].transpose(0, 2, 1),
                   ((0, 0), (0, 0), (0, 128 - TAIL)))
    outT = _embed_sum(xt, tabT, tail)
    return outT.T


# speedup vs baseline: 1.3369x; 1.0834x over previous
import jax
import jax.numpy as jnp
from jax import lax
from jax.experimental import pallas as pl
from jax.experimental.pallas import tpu as pltpu
from jax.experimental.pallas import tpu_sc as plsc

NUM_FIELDS = 26
VOCAB = 100000
EMBED_DIM = 32
BATCH = 16384
SCALE = NUM_FIELDS ** -0.5

_info = plsc.get_sparse_core_info()
NC, NS, L = _info.num_cores, _info.num_subcores, _info.num_lanes
NW = NC * NS
H0 = 50048
H1 = VOCAB - H0
TAIL = VOCAB % 128
H1MAIN = H1 - TAIL
CLAMPS = (H0, H1)
IC = 4096
NIC = BATCH // IC
UNROLL = 4


def _body(xt_hbm, tabT_hbm, tail_hbm, outT_hbm, idx0_v, idx1_v, idx2_v,
          row0_v, row1_v, acc_v, rsem0, rsem1, isem0, isem1, isem2):
    wid = lax.axis_index("s") * NC + lax.axis_index("c")
    idxs = (idx0_v, idx1_v, idx2_v)
    isems = (isem0, isem1, isem2)

    zeros = jnp.zeros((L,), jnp.float32)
    row0_v[pl.ds(H0, L)] = zeros

    @plsc.parallel_loop(0, BATCH, step=L, unroll=2)
    def _zero(i):
        acc_v[pl.ds(i, L)] = zeros

    def row_descs(f, h):
        if h == 0:
            return [pltpu.make_async_copy(
                tabT_hbm.at[f, wid, pl.ds(0, H0)],
                row0_v.at[pl.ds(0, H0)], rsem0)]
        return [
            pltpu.make_async_copy(
                tabT_hbm.at[f, wid, pl.ds(H0, H1MAIN)],
                row1_v.at[pl.ds(0, H1MAIN)], rsem1),
            pltpu.make_async_copy(
                tail_hbm.at[f, wid], row1_v.at[pl.ds(H1MAIN, 128)], rsem1),
        ]

    def row_issue(f, h):
        for d in row_descs(f, h):
            d.start()

    def row_wait(h):
        for d in row_descs(0, h):
            d.wait()

    def idx_issue(f, g):
        c = g % NIC
        pltpu.make_async_copy(
            xt_hbm.at[f, pl.ds(c * IC, IC)], idxs[g % 3], isems[g % 3]
        ).start()

    def idx_wait(g):
        pltpu.make_async_copy(
            xt_hbm.at[0, pl.ds(0, IC)], idxs[g % 3], isems[g % 3]).wait()

    row_issue(0, 0)
    row_issue(0, 1)
    idx_issue(0, 0)
    idx_issue(0, 1)
    idx_issue(0, 2)

    @pl.loop(0, NUM_FIELDS)
    def _field(f):
        not_last = f < NUM_FIELDS - 1
        for h in (0, 1):
            row_wait(h)
            for c in range(NIC):
                g = h * NIC + c
                idx_wait(g)
                ib = g % 3
                cbase = c * IC

                @plsc.parallel_loop(0, IC, step=L, unroll=UNROLL)
                def _gather(i):
                    iv = idxs[ib][pl.ds(i, L)]
                    if h:
                        iv = iv - H0
                    ivc = jnp.minimum(plsc.bitcast(iv, jnp.uint32),
                                      jnp.uint32(CLAMPS[h]))
                    vals = plsc.load_gather(
                        (row0_v, row1_v)[h],
                        [plsc.bitcast(ivc, jnp.int32)])
                    plsc.addupdate(acc_v.at[pl.ds(cbase + i, L)], vals)

                if g + 3 < 2 * NIC:
                    idx_issue(f, g + 3)
                else:
                    @pl.when(not_last)
                    def _pre_idx():
                        idx_issue(f + 1, g + 3 - 2 * NIC)

            @pl.when(not_last)
            def _pre_row():
                row_issue(f + 1, h)

    @plsc.parallel_loop(0, BATCH, step=L, unroll=2)
    def _scale(i):
        sl = pl.ds(i, L)
        acc_v[sl] = acc_v[sl] * SCALE

    pltpu.sync_copy(acc_v, outT_hbm.at[wid])


def _embed_sum(xt, tabT, tail):
    mesh = plsc.VectorSubcoreMesh(core_axis_name="c", subcore_axis_name="s")
    return pl.kernel(
        _body,
        out_type=jax.ShapeDtypeStruct((EMBED_DIM, BATCH), jnp.float32),
        mesh=mesh,
        scratch_types=[
            pltpu.VMEM((IC,), jnp.int32),
            pltpu.VMEM((IC,), jnp.int32),
            pltpu.VMEM((IC,), jnp.int32),
            pltpu.VMEM((H0 + L,), jnp.float32),
            pltpu.VMEM((H1MAIN + 128,), jnp.float32),
            pltpu.VMEM((BATCH,), jnp.float32),
            pltpu.SemaphoreType.DMA,
            pltpu.SemaphoreType.DMA,
            pltpu.SemaphoreType.DMA,
            pltpu.SemaphoreType.DMA,
            pltpu.SemaphoreType.DMA,
        ],
        compiler_params=pltpu.CompilerParams(needs_layout_passes=False),
    )(xt, tabT, tail)


def kernel(x, tables):
    if x.ndim == 1:
        x = x[:, None]
    xt = x.T
    tabT = tables.transpose(0, 2, 1)
    tail = jnp.pad(tables[:, VOCAB - TAIL:, :].transpose(0, 2, 1),
                   ((0, 0), (0, 0), (0, 128 - TAIL)))
    outT = _embed_sum(xt, tabT, tail)
    return outT.T
